# Initial kernel scaffold; baseline (speedup 1.0000x reference)
import jax
import jax.numpy as jnp
from jax.experimental import pallas as pl


def kernel(x, edge_index, Wl1, bl1, Wr1, Wl2, bl2, Wr2):
    raise NotImplementedError("write your pallas kernel here")


# speedup vs baseline: 5.2069x; 5.2069x over previous
import functools

import jax
import jax.numpy as jnp
from jax import lax
from jax.experimental import pallas as pl
from jax.experimental.pallas import tpu as pltpu
from jax.experimental.pallas import tpu_sc as plsc

N_NODES = 10000
N_EDGES = 320000
D = 128

NC = 2
NS = 16
NW = NC * NS

NP = 10240
EP = 327680
EPT = EP // NW
CHUNK = 128
NCHUNK = EPT // CHUNK
RPT = NP // NS


def _seg_body(x_hbm, src_hbm, dst_hbm, z_hbm, out_hbm, src_v, dst_v, rows_v, acc_sh):
    c = lax.axis_index("c")
    s = lax.axis_index("s")
    wid = c * NS + s

    pltpu.sync_copy(z_hbm.at[pl.ds(s * RPT, RPT)], acc_sh.at[pl.ds(s * RPT, RPT)])
    plsc.subcore_barrier()

    def chunk(j, carry):
        pltpu.sync_copy(src_hbm.at[pl.ds(wid * EPT + j * CHUNK, CHUNK)], src_v)
        pltpu.sync_copy(dst_hbm.at[pl.ds(wid * NCHUNK + j, 1)], dst_v)
        pltpu.sync_copy(x_hbm.at[src_v], rows_v)
        pltpu.sync_copy(rows_v, acc_sh.at[dst_v.at[0]], add=True)
        return carry

    lax.fori_loop(0, NCHUNK, chunk, 0)
    plsc.subcore_barrier()

    pltpu.sync_copy(acc_sh.at[pl.ds(s * RPT, RPT)], out_hbm.at[c, pl.ds(s * RPT, RPT)])


def _cnt_body(dst_hbm, z_hbm, ones_hbm, cnt_hbm, dst_v, ones_v, cnt_sh):
    c = lax.axis_index("c")
    s = lax.axis_index("s")
    wid = c * NS + s

    pltpu.sync_copy(z_hbm.at[pl.ds(s * RPT, RPT)], cnt_sh.at[pl.ds(s * RPT, RPT)])
    pltpu.sync_copy(ones_hbm, ones_v)
    plsc.subcore_barrier()

    def chunk(j, carry):
        pltpu.sync_copy(dst_hbm.at[pl.ds(wid * NCHUNK + j, 1)], dst_v)
        pltpu.sync_copy(ones_v, cnt_sh.at[dst_v.at[0]], add=True)
        return carry

    lax.fori_loop(0, NCHUNK, chunk, 0)
    plsc.subcore_barrier()

    pltpu.sync_copy(cnt_sh.at[pl.ds(s * RPT, RPT)], cnt_hbm.at[c, pl.ds(s * RPT, RPT)])


def _sc_mesh():
    return plsc.VectorSubcoreMesh(
        core_axis_name="c", subcore_axis_name="s", num_cores=NC, num_subcores=NS
    )


@functools.lru_cache(maxsize=None)
def _make_seg_sum():
    return pl.kernel(
        _seg_body,
        out_type=[jax.ShapeDtypeStruct((NC, NP, D), jnp.float32)],
        mesh=_sc_mesh(),
        scratch_types=[
            pltpu.VMEM((CHUNK,), jnp.int32),
            pltpu.VMEM((1, CHUNK), jnp.int32),
            pltpu.VMEM((CHUNK, D), jnp.float32),
            pltpu.VMEM_SHARED((NP, D), jnp.float32),
        ],
    )


@functools.lru_cache(maxsize=None)
def _make_counts():
    return pl.kernel(
        _cnt_body,
        out_type=[jax.ShapeDtypeStruct((NC, NP, D), jnp.float32)],
        mesh=_sc_mesh(),
        scratch_types=[
            pltpu.VMEM((1, CHUNK), jnp.int32),
            pltpu.VMEM((CHUNK, D), jnp.float32),
            pltpu.VMEM_SHARED((NP, D), jnp.float32),
        ],
    )


def _tc_body(relu, parts_ref, cnts_ref, x_ref, wl_ref, bl_ref, wr_ref, out_ref):
    summed = parts_ref[0] + parts_ref[1]
    cnt = cnts_ref[0, :, 0:1] + cnts_ref[1, :, 0:1]
    mean = summed * (1.0 / jnp.maximum(cnt, 1.0))
    h = (jnp.dot(mean, wl_ref[...], preferred_element_type=jnp.float32)
         + bl_ref[...]
         + jnp.dot(x_ref[...], wr_ref[...], preferred_element_type=jnp.float32))
    out_ref[...] = jnp.maximum(h, 0.0) if relu else h


BN = 512


def _sage_tc(parts, cnts, x, wl, bl, wr, relu):
    grid = NP // BN
    return pl.pallas_call(
        functools.partial(_tc_body, relu),
        grid=(grid,),
        in_specs=[
            pl.BlockSpec((NC, BN, D), lambda j: (0, j, 0)),
            pl.BlockSpec((NC, BN, D), lambda j: (0, j, 0)),
            pl.BlockSpec((BN, D), lambda j: (j, 0)),
            pl.BlockSpec((D, D), lambda j: (0, 0)),
            pl.BlockSpec((1, D), lambda j: (0, 0)),
            pl.BlockSpec((D, D), lambda j: (0, 0)),
        ],
        out_specs=pl.BlockSpec((BN, D), lambda j: (j, 0)),
        out_shape=jax.ShapeDtypeStruct((N_NODES, D), jnp.float32),
    )(parts, cnts, x, wl, bl.reshape(1, D), wr)


def kernel(x, edge_index, Wl1, bl1, Wr1, Wl2, bl2, Wr2):
    src = edge_index[0]
    dst = edge_index[1]
    pad = EP - N_EDGES
    pad_ids = jnp.arange(pad, dtype=jnp.int32)
    src_p = jnp.concatenate([src, pad_ids % N_NODES])
    dst_p = jnp.concatenate([dst, N_NODES + pad_ids % (NP - N_NODES)])
    dst2d = dst_p.reshape(EP // CHUNK, CHUNK)

    zeros_rows = jnp.zeros((NP, D), jnp.float32)
    ones_rows = jnp.ones((CHUNK, D), jnp.float32)

    cnts = _make_counts()(dst2d, zeros_rows, ones_rows)[0]
    parts1 = _make_seg_sum()(x, src_p, dst2d, zeros_rows)[0]
    h = _sage_tc(parts1, cnts, x, Wl1, bl1, Wr1, relu=True)
    parts2 = _make_seg_sum()(h, src_p, dst2d, zeros_rows)[0]
    out = _sage_tc(parts2, cnts, h, Wl2, bl2, Wr2, relu=False)
    return out


# speedup vs baseline: 7.8716x; 1.5118x over previous
import functools

import jax
import jax.numpy as jnp
from jax import lax
from jax.experimental import pallas as pl
from jax.experimental.pallas import tpu as pltpu
from jax.experimental.pallas import tpu_sc as plsc

N_NODES = 10000
N_EDGES = 320000
D = 128

NC = 2
NS = 16
NW = NC * NS

NP = 10240
EP = 327680
EPT = EP // NW
CHUNK = 64
NCHUNK = EPT // CHUNK
GRP = 16
NGRP = NCHUNK // GRP
CCH = 128
NCCH = EPT // CCH
CGRP = 8
NCGRP = NCCH // CGRP
RPT = NP // NS


def _seg_body(x_hbm, src_hbm, dst_hbm, z_hbm, out_hbm,
              src_g, dst_g, rows_a, rows_b, gs0, gs1, ss0, ss1, acc_sh):
    c = lax.axis_index("c")
    s = lax.axis_index("s")
    wid = c * NS + s
    rows = (rows_a, rows_b)
    gsem = (gs0, gs1)
    ssem = (ss0, ss1)

    pltpu.sync_copy(z_hbm.at[pl.ds(s * RPT, RPT)], acc_sh.at[pl.ds(s * RPT, RPT)])
    plsc.subcore_barrier()

    def group(g, carry):
        pltpu.sync_copy(src_hbm.at[pl.ds(wid * EPT + g * GRP * CHUNK, GRP * CHUNK)],
                        src_g)
        pltpu.sync_copy(dst_hbm.at[pl.ds(wid * NCHUNK + g * GRP, GRP)], dst_g)
        gd = [None, None]
        sd = [None, None]
        for b in range(GRP):
            p = b & 1
            if sd[p] is not None:
                sd[p].wait()
            gd[p] = pltpu.async_copy(
                x_hbm.at[src_g.at[pl.ds(b * CHUNK, CHUNK)]], rows[p], gsem[p])
            if b > 0:
                q = (b - 1) & 1
                gd[q].wait()
                sd[q] = pltpu.async_copy(
                    rows[q], acc_sh.at[dst_g.at[b - 1]], ssem[q], add=True)
        q = (GRP - 1) & 1
        gd[q].wait()
        sd[q] = pltpu.async_copy(
            rows[q], acc_sh.at[dst_g.at[GRP - 1]], ssem[q], add=True)
        sd[0].wait()
        sd[1].wait()
        return carry

    lax.fori_loop(0, NGRP, group, 0)
    plsc.subcore_barrier()

    pltpu.sync_copy(acc_sh.at[pl.ds(s * RPT, RPT)], out_hbm.at[c, pl.ds(s * RPT, RPT)])


def _cnt_body(dst_hbm, z_hbm, ones_hbm, cnt_hbm, dst_g, ones_v, cs, cnt_sh):
    c = lax.axis_index("c")
    s = lax.axis_index("s")
    wid = c * NS + s

    pltpu.sync_copy(z_hbm.at[pl.ds(s * RPT, RPT)], cnt_sh.at[pl.ds(s * RPT, RPT)])
    pltpu.sync_copy(ones_hbm, ones_v)
    plsc.subcore_barrier()

    def group(g, carry):
        pltpu.sync_copy(dst_hbm.at[pl.ds(wid * NCCH + g * CGRP, CGRP)], dst_g)
        descs = [
            pltpu.async_copy(ones_v, cnt_sh.at[dst_g.at[b]], cs, add=True)
            for b in range(CGRP)
        ]
        for d in descs:
            d.wait()
        return carry

    lax.fori_loop(0, NCGRP, group, 0)
    plsc.subcore_barrier()

    pltpu.sync_copy(cnt_sh.at[pl.ds(s * RPT, RPT)], cnt_hbm.at[c, pl.ds(s * RPT, RPT)])


def _sc_mesh():
    return plsc.VectorSubcoreMesh(
        core_axis_name="c", subcore_axis_name="s", num_cores=NC, num_subcores=NS
    )


@functools.lru_cache(maxsize=None)
def _make_seg_sum():
    return pl.kernel(
        _seg_body,
        out_type=[jax.ShapeDtypeStruct((NC, NP, D), jnp.float32)],
        mesh=_sc_mesh(),
        scratch_types=[
            pltpu.VMEM((GRP * CHUNK,), jnp.int32),
            pltpu.VMEM((GRP, CHUNK), jnp.int32),
            pltpu.VMEM((CHUNK, D), jnp.float32),
            pltpu.VMEM((CHUNK, D), jnp.float32),
            pltpu.SemaphoreType.DMA,
            pltpu.SemaphoreType.DMA,
            pltpu.SemaphoreType.DMA,
            pltpu.SemaphoreType.DMA,
            pltpu.VMEM_SHARED((NP, D), jnp.float32),
        ],
    )


@functools.lru_cache(maxsize=None)
def _make_counts():
    return pl.kernel(
        _cnt_body,
        out_type=[jax.ShapeDtypeStruct((NC, NP, D), jnp.float32)],
        mesh=_sc_mesh(),
        scratch_types=[
            pltpu.VMEM((CGRP, CCH), jnp.int32),
            pltpu.VMEM((CCH, D), jnp.float32),
            pltpu.SemaphoreType.DMA,
            pltpu.VMEM_SHARED((NP, D), jnp.float32),
        ],
    )


def _tc_body(relu, parts_ref, cnts_ref, x_ref, wl_ref, bl_ref, wr_ref, out_ref):
    summed = parts_ref[0] + parts_ref[1]
    cnt = cnts_ref[0, :, 0:1] + cnts_ref[1, :, 0:1]
    mean = summed * (1.0 / jnp.maximum(cnt, 1.0))
    h = (jnp.dot(mean, wl_ref[...], preferred_element_type=jnp.float32)
         + bl_ref[...]
         + jnp.dot(x_ref[...], wr_ref[...], preferred_element_type=jnp.float32))
    out_ref[...] = jnp.maximum(h, 0.0) if relu else h


BN = 512


def _sage_tc(parts, cnts, x, wl, bl, wr, relu):
    grid = NP // BN
    return pl.pallas_call(
        functools.partial(_tc_body, relu),
        grid=(grid,),
        in_specs=[
            pl.BlockSpec((NC, BN, D), lambda j: (0, j, 0)),
            pl.BlockSpec((NC, BN, D), lambda j: (0, j, 0)),
            pl.BlockSpec((BN, D), lambda j: (j, 0)),
            pl.BlockSpec((D, D), lambda j: (0, 0)),
            pl.BlockSpec((1, D), lambda j: (0, 0)),
            pl.BlockSpec((D, D), lambda j: (0, 0)),
        ],
        out_specs=pl.BlockSpec((BN, D), lambda j: (j, 0)),
        out_shape=jax.ShapeDtypeStruct((N_NODES, D), jnp.float32),
    )(parts, cnts, x, wl, bl.reshape(1, D), wr)


def kernel(x, edge_index, Wl1, bl1, Wr1, Wl2, bl2, Wr2):
    src = edge_index[0]
    dst = edge_index[1]
    pad = EP - N_EDGES
    pad_ids = jnp.arange(pad, dtype=jnp.int32)
    src_p = jnp.concatenate([src, pad_ids % N_NODES])
    dst_p = jnp.concatenate([dst, N_NODES + pad_ids % (NP - N_NODES)])
    dst2d = dst_p.reshape(EP // CHUNK, CHUNK)
    dst2dc = dst_p.reshape(EP // CCH, CCH)

    zeros_rows = jnp.zeros((NP, D), jnp.float32)
    ones_rows = jnp.ones((CCH, D), jnp.float32)

    cnts = _make_counts()(dst2dc, zeros_rows, ones_rows)[0]
    parts1 = _make_seg_sum()(x, src_p, dst2d, zeros_rows)[0]
    h = _sage_tc(parts1, cnts, x, Wl1, bl1, Wr1, relu=True)
    parts2 = _make_seg_sum()(h, src_p, dst2d, zeros_rows)[0]
    out = _sage_tc(parts2, cnts, h, Wl2, bl2, Wr2, relu=False)
    return out


# speedup vs baseline: 8.4253x; 1.0704x over previous
import functools

import jax
import jax.numpy as jnp
from jax import lax
from jax.experimental import pallas as pl
from jax.experimental.pallas import tpu as pltpu
from jax.experimental.pallas import tpu_sc as plsc

N_NODES = 10000
N_EDGES = 320000
D = 128

NC = 2
NS = 16
NW = NC * NS

NP = 10240
EP = 327680
EPT = EP // NW
CHUNK = 64
NCHUNK = EPT // CHUNK
CCH = 128
NCCH = EPT // CCH
RPT = NP // NS


def _seg_body(x_hbm, src_hbm, dst_hbm, z_hbm, out_hbm,
              src_g, dst_g, rows_a, rows_b, gs0, gs1, ss0, ss1, acc_sh):
    c = lax.axis_index("c")
    s = lax.axis_index("s")
    wid = c * NS + s
    rows = (rows_a, rows_b)
    gsem = (gs0, gs1)
    ssem = (ss0, ss1)

    pltpu.sync_copy(z_hbm.at[pl.ds(s * RPT, RPT)], acc_sh.at[pl.ds(s * RPT, RPT)])
    plsc.subcore_barrier()

    pltpu.sync_copy(src_hbm.at[pl.ds(wid * EPT, EPT)], src_g)
    pltpu.sync_copy(dst_hbm.at[pl.ds(wid * NCHUNK, NCHUNK)], dst_g)
    gd = [None, None]
    sd = [None, None]
    for b in range(NCHUNK):
        p = b & 1
        if sd[p] is not None:
            sd[p].wait()
        gd[p] = pltpu.async_copy(
            x_hbm.at[src_g.at[pl.ds(b * CHUNK, CHUNK)]], rows[p], gsem[p])
        if b > 0:
            q = (b - 1) & 1
            gd[q].wait()
            sd[q] = pltpu.async_copy(
                rows[q], acc_sh.at[dst_g.at[b - 1]], ssem[q], add=True)
    q = (NCHUNK - 1) & 1
    gd[q].wait()
    sd[q] = pltpu.async_copy(
        rows[q], acc_sh.at[dst_g.at[NCHUNK - 1]], ssem[q], add=True)
    sd[0].wait()
    sd[1].wait()
    plsc.subcore_barrier()

    pltpu.sync_copy(acc_sh.at[pl.ds(s * RPT, RPT)], out_hbm.at[c, pl.ds(s * RPT, RPT)])


def _cnt_body(dst_hbm, z_hbm, ones_hbm, cnt_hbm, dst_g, ones_v, cs, cnt_sh):
    c = lax.axis_index("c")
    s = lax.axis_index("s")
    wid = c * NS + s

    pltpu.sync_copy(z_hbm.at[pl.ds(s * RPT, RPT)], cnt_sh.at[pl.ds(s * RPT, RPT)])
    pltpu.sync_copy(ones_hbm, ones_v)
    plsc.subcore_barrier()

    pltpu.sync_copy(dst_hbm.at[pl.ds(wid * NCCH, NCCH)], dst_g)
    descs = []
    for b in range(NCCH):
        if b >= 8:
            descs[b - 8].wait()
        descs.append(
            pltpu.async_copy(ones_v, cnt_sh.at[dst_g.at[b]], cs, add=True))
    for d in descs[-8:]:
        d.wait()
    plsc.subcore_barrier()

    pltpu.sync_copy(cnt_sh.at[pl.ds(s * RPT, RPT)], cnt_hbm.at[c, pl.ds(s * RPT, RPT)])


def _sc_mesh():
    return plsc.VectorSubcoreMesh(
        core_axis_name="c", subcore_axis_name="s", num_cores=NC, num_subcores=NS
    )


@functools.lru_cache(maxsize=None)
def _make_seg_sum():
    return pl.kernel(
        _seg_body,
        out_type=[jax.ShapeDtypeStruct((NC, NP, D), jnp.float32)],
        mesh=_sc_mesh(),
        scratch_types=[
            pltpu.VMEM((EPT,), jnp.int32),
            pltpu.VMEM((NCHUNK, CHUNK), jnp.int32),
            pltpu.VMEM((CHUNK, D), jnp.float32),
            pltpu.VMEM((CHUNK, D), jnp.float32),
            pltpu.SemaphoreType.DMA,
            pltpu.SemaphoreType.DMA,
            pltpu.SemaphoreType.DMA,
            pltpu.SemaphoreType.DMA,
            pltpu.VMEM_SHARED((NP, D), jnp.float32),
        ],
    )


@functools.lru_cache(maxsize=None)
def _make_counts():
    return pl.kernel(
        _cnt_body,
        out_type=[jax.ShapeDtypeStruct((NC, NP, D), jnp.float32)],
        mesh=_sc_mesh(),
        scratch_types=[
            pltpu.VMEM((NCCH, CCH), jnp.int32),
            pltpu.VMEM((CCH, D), jnp.float32),
            pltpu.SemaphoreType.DMA,
            pltpu.VMEM_SHARED((NP, D), jnp.float32),
        ],
    )


def _tc_body(relu, parts_ref, cnts_ref, x_ref, wl_ref, bl_ref, wr_ref, out_ref):
    summed = parts_ref[0] + parts_ref[1]
    cnt = cnts_ref[0, :, 0:1] + cnts_ref[1, :, 0:1]
    mean = summed * (1.0 / jnp.maximum(cnt, 1.0))
    h = (jnp.dot(mean, wl_ref[...], preferred_element_type=jnp.float32)
         + bl_ref[...]
         + jnp.dot(x_ref[...], wr_ref[...], preferred_element_type=jnp.float32))
    out_ref[...] = jnp.maximum(h, 0.0) if relu else h


BN = 512


def _sage_tc(parts, cnts, x, wl, bl, wr, relu):
    grid = NP // BN
    return pl.pallas_call(
        functools.partial(_tc_body, relu),
        grid=(grid,),
        in_specs=[
            pl.BlockSpec((NC, BN, D), lambda j: (0, j, 0)),
            pl.BlockSpec((NC, BN, D), lambda j: (0, j, 0)),
            pl.BlockSpec((BN, D), lambda j: (j, 0)),
            pl.BlockSpec((D, D), lambda j: (0, 0)),
            pl.BlockSpec((1, D), lambda j: (0, 0)),
            pl.BlockSpec((D, D), lambda j: (0, 0)),
        ],
        out_specs=pl.BlockSpec((BN, D), lambda j: (j, 0)),
        out_shape=jax.ShapeDtypeStruct((N_NODES, D), jnp.float32),
    )(parts, cnts, x, wl, bl.reshape(1, D), wr)


def kernel(x, edge_index, Wl1, bl1, Wr1, Wl2, bl2, Wr2):
    src = edge_index[0]
    dst = edge_index[1]
    pad = EP - N_EDGES
    pad_ids = jnp.arange(pad, dtype=jnp.int32)
    src_p = jnp.concatenate([src, pad_ids % N_NODES])
    dst_p = jnp.concatenate([dst, N_NODES + pad_ids % (NP - N_NODES)])
    dst2d = dst_p.reshape(EP // CHUNK, CHUNK)
    dst2dc = dst_p.reshape(EP // CCH, CCH)

    zeros_rows = jnp.zeros((NP, D), jnp.float32)
    ones_rows = jnp.ones((CCH, D), jnp.float32)

    cnts = _make_counts()(dst2dc, zeros_rows, ones_rows)[0]
    parts1 = _make_seg_sum()(x, src_p, dst2d, zeros_rows)[0]
    h = _sage_tc(parts1, cnts, x, Wl1, bl1, Wr1, relu=True)
    parts2 = _make_seg_sum()(h, src_p, dst2d, zeros_rows)[0]
    out = _sage_tc(parts2, cnts, h, Wl2, bl2, Wr2, relu=False)
    return out


# speedup vs baseline: 9.1475x; 1.0857x over previous
import functools

import jax
import jax.numpy as jnp
from jax import lax
from jax.experimental import pallas as pl
from jax.experimental.pallas import tpu as pltpu
from jax.experimental.pallas import tpu_sc as plsc

N_NODES = 10000
N_EDGES = 320000
D = 128

NC = 2
NS = 16
NW = NC * NS

NP = 10240
EP = 327680
EPT = EP // NW
CHUNK = 128
NCHUNK = EPT // CHUNK
GRP = 8
NGRP = NCHUNK // GRP
RPT = NP // NS


def _seg_body(x_hbm, src_hbm, dst_hbm, z_hbm, out_hbm,
              src_g, dst_g, rows_a, rows_b, gs0, gs1, ss0, ss1, acc_sh):
    c = lax.axis_index("c")
    s = lax.axis_index("s")
    wid = c * NS + s
    rows = (rows_a, rows_b)
    gsem = (gs0, gs1)
    ssem = (ss0, ss1)

    pltpu.sync_copy(z_hbm.at[pl.ds(s * RPT, RPT)], acc_sh.at[pl.ds(s * RPT, RPT)])
    plsc.subcore_barrier()

    def stage(g):
        e = g & 1
        pltpu.sync_copy(
            src_hbm.at[pl.ds(wid * EPT + g * GRP * CHUNK, GRP * CHUNK)],
            src_g.at[e])
        pltpu.sync_copy(dst_hbm.at[pl.ds(wid * NCHUNK + g * GRP, GRP)],
                        dst_g.at[e])

    stage(0)
    gd = [None, None]
    sd = [None, None]
    for g in range(NGRP):
        if g + 1 < NGRP:
            stage(g + 1)
        for i in range(GRP):
            b = g * GRP + i
            p = b & 1
            if sd[p] is not None:
                sd[p].wait()
            gd[p] = pltpu.async_copy(
                x_hbm.at[src_g.at[g & 1, pl.ds(i * CHUNK, CHUNK)]],
                rows[p], gsem[p])
            if b > 0:
                bp = b - 1
                q = bp & 1
                gd[q].wait()
                sd[q] = pltpu.async_copy(
                    rows[q], acc_sh.at[dst_g.at[(bp // GRP) & 1, bp % GRP]],
                    ssem[q], add=True)
    bp = NCHUNK - 1
    q = bp & 1
    gd[q].wait()
    sd[q] = pltpu.async_copy(
        rows[q], acc_sh.at[dst_g.at[(bp // GRP) & 1, bp % GRP]], ssem[q], add=True)
    sd[0].wait()
    sd[1].wait()
    plsc.subcore_barrier()

    pltpu.sync_copy(acc_sh.at[pl.ds(s * RPT, RPT)], out_hbm.at[c, pl.ds(s * RPT, RPT)])


def _cnt_body(dst_hbm, z_hbm, ones_hbm, cnt_hbm, dst_g, ones_v, cs, cnt_sh):
    c = lax.axis_index("c")
    s = lax.axis_index("s")
    wid = c * NS + s

    pltpu.sync_copy(z_hbm.at[pl.ds(s * RPT, RPT)], cnt_sh.at[pl.ds(s * RPT, RPT)])
    pltpu.sync_copy(ones_hbm, ones_v)
    plsc.subcore_barrier()

    pltpu.sync_copy(dst_hbm.at[pl.ds(wid * NCHUNK, NCHUNK)], dst_g)
    descs = []
    for b in range(NCHUNK):
        if b >= 8:
            descs[b - 8].wait()
        descs.append(
            pltpu.async_copy(ones_v, cnt_sh.at[dst_g.at[b]], cs, add=True))
    for d in descs[-8:]:
        d.wait()
    plsc.subcore_barrier()

    pltpu.sync_copy(cnt_sh.at[pl.ds(s * RPT, RPT)], cnt_hbm.at[c, pl.ds(s * RPT, RPT)])


def _sc_mesh():
    return plsc.VectorSubcoreMesh(
        core_axis_name="c", subcore_axis_name="s", num_cores=NC, num_subcores=NS
    )


@functools.lru_cache(maxsize=None)
def _make_seg_sum():
    return pl.kernel(
        _seg_body,
        out_type=[jax.ShapeDtypeStruct((NC, NP, D), jnp.float32)],
        mesh=_sc_mesh(),
        scratch_types=[
            pltpu.VMEM((2, GRP * CHUNK), jnp.int32),
            pltpu.VMEM((2, GRP, CHUNK), jnp.int32),
            pltpu.VMEM((CHUNK, D), jnp.float32),
            pltpu.VMEM((CHUNK, D), jnp.float32),
            pltpu.SemaphoreType.DMA,
            pltpu.SemaphoreType.DMA,
            pltpu.SemaphoreType.DMA,
            pltpu.SemaphoreType.DMA,
            pltpu.VMEM_SHARED((NP, D), jnp.float32),
        ],
    )


@functools.lru_cache(maxsize=None)
def _make_counts():
    return pl.kernel(
        _cnt_body,
        out_type=[jax.ShapeDtypeStruct((NC, NP, D), jnp.float32)],
        mesh=_sc_mesh(),
        scratch_types=[
            pltpu.VMEM((NCHUNK, CHUNK), jnp.int32),
            pltpu.VMEM((CHUNK, D), jnp.float32),
            pltpu.SemaphoreType.DMA,
            pltpu.VMEM_SHARED((NP, D), jnp.float32),
        ],
    )


def _tc_body(relu, parts_ref, cnts_ref, x_ref, wl_ref, bl_ref, wr_ref, out_ref):
    summed = parts_ref[0] + parts_ref[1]
    cnt = cnts_ref[0] + cnts_ref[1]
    mean = summed * (1.0 / jnp.maximum(cnt, 1.0))
    h = (jnp.dot(mean, wl_ref[...], preferred_element_type=jnp.float32)
         + bl_ref[...]
         + jnp.dot(x_ref[...], wr_ref[...], preferred_element_type=jnp.float32))
    out_ref[...] = jnp.maximum(h, 0.0) if relu else h


BN = 512


def _sage_tc(parts, cnts, x, wl, bl, wr, relu):
    grid = NP // BN
    return pl.pallas_call(
        functools.partial(_tc_body, relu),
        grid=(grid,),
        in_specs=[
            pl.BlockSpec((NC, BN, D), lambda j: (0, j, 0)),
            pl.BlockSpec((NC, BN, 1), lambda j: (0, j, 0)),
            pl.BlockSpec((BN, D), lambda j: (j, 0)),
            pl.BlockSpec((D, D), lambda j: (0, 0)),
            pl.BlockSpec((1, D), lambda j: (0, 0)),
            pl.BlockSpec((D, D), lambda j: (0, 0)),
        ],
        out_specs=pl.BlockSpec((BN, D), lambda j: (j, 0)),
        out_shape=jax.ShapeDtypeStruct((N_NODES, D), jnp.float32),
    )(parts, cnts, x, wl, bl.reshape(1, D), wr)


def kernel(x, edge_index, Wl1, bl1, Wr1, Wl2, bl2, Wr2):
    src = edge_index[0]
    dst = edge_index[1]
    pad = EP - N_EDGES
    pad_ids = jnp.arange(pad, dtype=jnp.int32)
    src_p = jnp.concatenate([src, pad_ids % N_NODES])
    dst_p = jnp.concatenate([dst, N_NODES + pad_ids % (NP - N_NODES)])
    dst2d = dst_p.reshape(EP // CHUNK, CHUNK)

    zeros_rows = jnp.zeros((NP, D), jnp.float32)
    ones_rows = jnp.ones((CHUNK, D), jnp.float32)

    cnts = _make_counts()(dst2d, zeros_rows, ones_rows)[0][:, :, 0:1]
    parts1 = _make_seg_sum()(x, src_p, dst2d, zeros_rows)[0]
    h = _sage_tc(parts1, cnts, x, Wl1, bl1, Wr1, relu=True)
    parts2 = _make_seg_sum()(h, src_p, dst2d, zeros_rows)[0]
    out = _sage_tc(parts2, cnts, h, Wl2, bl2, Wr2, relu=False)
    return out
